# Initial kernel scaffold; baseline (speedup 1.0000x reference)
import jax
import jax.numpy as jnp
from jax.experimental import pallas as pl


def kernel(h, edge_index, norm, W, b):
    raise NotImplementedError("write your pallas kernel here")


# speedup vs baseline: 6.9062x; 6.9062x over previous
import functools

import jax
import jax.numpy as jnp
from jax import lax
from jax.experimental import pallas as pl
from jax.experimental.pallas import tpu as pltpu
from jax.experimental.pallas import tpu_sc as plsc

N_NODES = 10000
N_EDGES = 160000
IN_F = 512
OUT_F = 256
HALF_F = 128
NC, NS = 2, 16
EPT = N_EDGES // NS
CHUNK = 80
NCHUNK = EPT // CHUNK
ROWS_PT = 632
N_PAD = ROWS_PT * NS
M_BLK = 1000


def _matmul_body(h_ref, w_ref, n_ref, out_ref):
    out_ref[0] = (
        jnp.dot(h_ref[...], w_ref[...], preferred_element_type=jnp.float32)
        * n_ref[...]
    )


def _projn(h, W, norm2):
    return pl.pallas_call(
        _matmul_body,
        grid=(N_NODES // M_BLK, NC),
        in_specs=[
            pl.BlockSpec((M_BLK, IN_F), lambda i, j: (i, 0)),
            pl.BlockSpec((IN_F, HALF_F), lambda i, j: (0, j)),
            pl.BlockSpec((M_BLK, 1), lambda i, j: (i, 0)),
        ],
        out_specs=pl.BlockSpec((1, M_BLK, HALF_F), lambda i, j: (j, i, 0)),
        out_shape=jax.ShapeDtypeStruct((NC, N_NODES, HALF_F), jnp.float32),
    )(h, W, norm2)


def _sc_aggregate(projn, src3, dst3, zeros):
    mesh = plsc.VectorSubcoreMesh(
        core_axis_name="c", subcore_axis_name="s", num_cores=NC, num_subcores=NS
    )

    @functools.partial(
        pl.kernel,
        out_type=jax.ShapeDtypeStruct((NC, N_PAD, HALF_F), jnp.float32),
        mesh=mesh,
        scratch_types=[
            pltpu.VMEM((NCHUNK, CHUNK), jnp.int32),
            pltpu.VMEM((NCHUNK, CHUNK), jnp.int32),
            pltpu.VMEM((CHUNK, HALF_F), jnp.float32),
            pltpu.VMEM_SHARED((N_PAD, HALF_F), jnp.float32),
            pltpu.SemaphoreType.DMA,
        ],
    )
    def k(projn_hbm, src_hbm, dst_hbm, zeros_hbm, out_hbm,
          src_v, dst_v, rows_v, acc, sem):
        c = lax.axis_index("c")
        s = lax.axis_index("s")
        pltpu.sync_copy(zeros_hbm, acc.at[pl.ds(s * ROWS_PT, ROWS_PT)])
        pltpu.sync_copy(src_hbm.at[s], src_v)
        pltpu.sync_copy(dst_hbm.at[s], dst_v)
        plsc.subcore_barrier()

        table = projn_hbm.at[c]

        def body(j, carry):
            pltpu.async_copy(table.at[src_v.at[j]], rows_v, sem).wait()
            pltpu.sync_copy(rows_v, acc.at[dst_v.at[j]], add=True)
            return carry

        lax.fori_loop(0, NCHUNK, body, 0)
        plsc.subcore_barrier()
        pltpu.sync_copy(
            acc.at[pl.ds(s * ROWS_PT, ROWS_PT)],
            out_hbm.at[c].at[pl.ds(s * ROWS_PT, ROWS_PT)],
        )

    return k(projn, src3, dst3, zeros)


def _epilogue_body(agg_ref, n_ref, b_ref, out_ref):
    out_ref[...] = agg_ref[0] * n_ref[...] + b_ref[...]


def _epilogue(agg, norm2, b2):
    return pl.pallas_call(
        _epilogue_body,
        grid=(N_NODES // M_BLK, NC),
        in_specs=[
            pl.BlockSpec((1, M_BLK, HALF_F), lambda i, j: (j, i, 0)),
            pl.BlockSpec((M_BLK, 1), lambda i, j: (i, 0)),
            pl.BlockSpec((1, HALF_F), lambda i, j: (0, j)),
        ],
        out_specs=pl.BlockSpec((M_BLK, HALF_F), lambda i, j: (i, j)),
        out_shape=jax.ShapeDtypeStruct((N_NODES, OUT_F), jnp.float32),
    )(agg, norm2, b2)


def kernel(h, edge_index, norm, W, b):
    src = edge_index[0].astype(jnp.int32)
    dst = edge_index[1].astype(jnp.int32)
    src3 = src.reshape(NS, NCHUNK, CHUNK)
    dst3 = dst.reshape(NS, NCHUNK, CHUNK)
    norm2 = norm.reshape(N_NODES, 1)
    zeros = jnp.zeros((ROWS_PT, HALF_F), jnp.float32)

    projn = _projn(h, W, norm2)
    agg = _sc_aggregate(projn, src3, dst3, zeros)
    return _epilogue(agg, norm2, b.reshape(1, OUT_F))
